# baseline (speedup 1.0000x reference)
import functools

import jax
import jax.numpy as jnp
import numpy as np
from jax import lax
from jax.experimental import pallas as pl
from jax.experimental.pallas import tpu as pltpu
from jax.experimental.pallas import tpu_sc as plsc

K = 20
SK = 10
NT = 256
SC_NC = 2
SC_NS = 16
SC_CH = 128

_SQ = np.sqrt(np.float32(1.0) + np.float32(1e-5)).astype(np.float32)
_NEG_INF = np.float32(-np.inf)


def _leaky(v):
    return jnp.where(v >= 0, v, np.float32(0.2) * v)


def _knn_body(x_ref, xt_ref, p1_ref, p2_ref, p3_ref, s1_ref, s2_ref, s3_ref):
    xb = x_ref[0]
    xt = xt_ref[0]
    c_dim, n = xb.shape

    s = jnp.dot(xt, xb, preferred_element_type=jnp.float32)
    xxr = xb[0:1, :] * xb[0:1, :]
    xxc = xt[:, 0:1] * xt[:, 0:1]
    for c in range(1, c_dim):
        xxr = xxr + xb[c:c + 1, :] * xb[c:c + 1, :]
        xxc = xxc + xt[:, c:c + 1] * xt[:, c:c + 1]
    inner = np.float32(-2.0) * s
    pd = (-xxc) - inner - xxr

    iota = jax.lax.broadcasted_iota(jnp.int32, (n, n), 0)
    p1 = p1_ref[0]
    p2 = p2_ref[0]
    p3 = p3_ref[0]
    s1 = jnp.zeros((SK, n), jnp.int32)
    s2 = jnp.zeros((SK, n), jnp.int32)
    s3 = jnp.zeros((SK, n), jnp.int32)
    for r in range(K):
        v, ids = pd, iota
        rows = n
        while rows > 1:
            hlf = rows // 2
            gt = v[:hlf] >= v[hlf:rows]
            v = jnp.where(gt, v[:hlf], v[hlf:rows])
            ids = jnp.where(gt, ids[:hlf], ids[hlf:rows])
            rows = hlf
        ai = ids
        s1 = jnp.where(p1 == r, ai, s1)
        s2 = jnp.where(p2 == r, ai, s2)
        s3 = jnp.where(p3 == r, ai, s3)
        if r < K - 1:
            pd = jnp.where(iota == ai, _NEG_INF, pd)
    base = pl.program_id(0) * n
    s1_ref[0] = s1 + base
    s2_ref[0] = s2 + base
    s3_ref[0] = s3 + base


def _knn_call(x, xt, p1, p2, p3):
    b, c, n = x.shape
    spec_sidx = pl.BlockSpec((1, SK, n), lambda i: (i, 0, 0))
    return pl.pallas_call(
        _knn_body,
        grid=(b,),
        in_specs=[
            pl.BlockSpec((1, c, n), lambda i: (i, 0, 0)),
            pl.BlockSpec((1, n, c), lambda i: (i, 0, 0)),
            spec_sidx, spec_sidx, spec_sidx,
        ],
        out_specs=[spec_sidx, spec_sidx, spec_sidx],
        out_shape=[jax.ShapeDtypeStruct((b, SK, n), jnp.int32)] * 3,
    )(x, xt, p1, p2, p3)


def _sc_gather(table, idx):
    m, = idx.shape
    d = table.shape[1]
    nw = SC_NC * SC_NS
    per_w = m // nw
    n_ch = per_w // SC_CH
    half = n_ch // 2
    mesh = plsc.VectorSubcoreMesh(core_axis_name="c", subcore_axis_name="s")

    @functools.partial(
        pl.kernel, mesh=mesh,
        compiler_params=pltpu.CompilerParams(use_tc_tiling_on_sc=False),
        out_type=jax.ShapeDtypeStruct((m, d), jnp.float32),
        scratch_types=[
            pltpu.VMEM((SC_CH,), jnp.int32),
            pltpu.VMEM((SC_CH,), jnp.int32),
            pltpu.VMEM((SC_CH, d), jnp.float32),
            pltpu.VMEM((SC_CH, d), jnp.float32),
            pltpu.SemaphoreType.DMA,
            pltpu.SemaphoreType.DMA,
        ],
    )
    def k(table_hbm, idx_hbm, out_hbm, idx0, idx1, rows0, rows1,
          sem0, sem1):
        wid = lax.axis_index("s") * SC_NC + lax.axis_index("c")
        base = wid * per_w

        pltpu.sync_copy(idx_hbm.at[pl.ds(base, SC_CH)], idx0)
        pltpu.async_copy(table_hbm.at[idx0], rows0, sem0)

        @pl.loop(0, half)
        def _(g):
            off = base + (2 * g) * SC_CH
            pltpu.sync_copy(idx_hbm.at[pl.ds(off + SC_CH, SC_CH)], idx1)
            pltpu.async_copy(table_hbm.at[idx1], rows1, sem1)
            pltpu.make_async_copy(table_hbm.at[idx0], rows0, sem0).wait()
            pltpu.sync_copy(rows0, out_hbm.at[pl.ds(off, SC_CH)])

            @pl.when(g < half - 1)
            def _():
                pltpu.sync_copy(
                    idx_hbm.at[pl.ds(off + 2 * SC_CH, SC_CH)], idx0)
                pltpu.async_copy(table_hbm.at[idx0], rows0, sem0)

            pltpu.make_async_copy(table_hbm.at[idx1], rows1, sem1).wait()
            pltpu.sync_copy(rows1, out_hbm.at[pl.ds(off + SC_CH, SC_CH)])

    return k(table, idx)


def _conv_body(c, paired, feat_ref, xc_ref, wda_ref, wdb_ref, wpa_ref,
               wpb_ref, *rest):
    if paired:
        par_ref, out_ref = rest
    else:
        (out_ref,) = rest
    xc = xc_ref[0]
    wda = wda_ref[...]
    wdb = wdb_ref[...]
    wpa = wpa_ref[...]
    wpb = wpb_ref[...]
    if paired:
        par = par_ref[0]

    term2 = jnp.dot(_leaky((xc * wdb) / _SQ), wpb,
                    preferred_element_type=jnp.float32)
    acc = jnp.full((NT, wpa.shape[1]), _NEG_INF, jnp.float32)
    for s in range(SK):
        fp = feat_ref[s]
        if paired:
            feat = jnp.where(par[:, s:s + 1] != 0, fp[:, c:2 * c],
                             fp[:, :c])
        else:
            feat = fp[:, :c]
        t1 = jnp.dot(_leaky(((feat - xc) * wda) / _SQ), wpa,
                     preferred_element_type=jnp.float32)
        acc = jnp.maximum(acc, t1)
    out_ref[0] = acc + term2


def _conv_call(feat, xc, c, w_dw, w_pw, par=None):
    b, n = xc.shape[0], xc.shape[1]
    d = feat.shape[2]
    o = w_pw.shape[0]
    wda = w_dw[:c].reshape(1, c)
    wdb = w_dw[c:].reshape(1, c)
    wpa = w_pw[:, :c].T
    wpb = w_pw[:, c:].T
    in_specs = [
        pl.BlockSpec((SK, NT, d), lambda i, j: (i, j, 0)),
        pl.BlockSpec((1, NT, c), lambda i, j: (i, j, 0)),
        pl.BlockSpec((1, c), lambda i, j: (0, 0)),
        pl.BlockSpec((1, c), lambda i, j: (0, 0)),
        pl.BlockSpec((c, o), lambda i, j: (0, 0)),
        pl.BlockSpec((c, o), lambda i, j: (0, 0)),
    ]
    args = [feat, xc, wda, wdb, wpa, wpb]
    if par is not None:
        in_specs.append(pl.BlockSpec((1, NT, SK), lambda i, j: (i, j, 0)))
        args.append(par)
    return pl.pallas_call(
        functools.partial(_conv_body, c, par is not None),
        grid=(b, n // NT),
        in_specs=in_specs,
        out_specs=pl.BlockSpec((1, NT, o), lambda i, j: (i, j, 0)),
        out_shape=jax.ShapeDtypeStruct((b, n, o), jnp.float32),
    )(*args)


def _fuse_body(f1_ref, f2_ref, f3_ref, wa_ref, wb_ref, wc_ref, h_ref):
    xe = jnp.dot(f1_ref[0], wa_ref[...], preferred_element_type=jnp.float32)
    xe = xe + jnp.dot(f2_ref[0], wb_ref[...],
                      preferred_element_type=jnp.float32)
    xe = xe + jnp.dot(f3_ref[0], wc_ref[...],
                      preferred_element_type=jnp.float32)
    xe = _leaky(xe / _SQ)
    x1 = jnp.max(xe, axis=0, keepdims=True)
    x2 = jnp.sum(xe, axis=0, keepdims=True) / np.float32(xe.shape[0])
    h_ref[0] = jnp.concatenate([x1, x2], axis=1)


def _fuse_call(f1, f2, f3, w_final):
    b, n, c1 = f1.shape
    c3 = f3.shape[2]
    o = w_final.shape[0]
    wa = w_final[:, :c1].T
    wb = w_final[:, c1:2 * c1].T
    wc = w_final[:, 2 * c1:].T
    return pl.pallas_call(
        _fuse_body,
        grid=(b,),
        in_specs=[
            pl.BlockSpec((1, n, c1), lambda i: (i, 0, 0)),
            pl.BlockSpec((1, n, c1), lambda i: (i, 0, 0)),
            pl.BlockSpec((1, n, c3), lambda i: (i, 0, 0)),
            pl.BlockSpec((c1, o), lambda i: (0, 0)),
            pl.BlockSpec((c1, o), lambda i: (0, 0)),
            pl.BlockSpec((c3, o), lambda i: (0, 0)),
        ],
        out_specs=pl.BlockSpec((1, 1, 2 * o), lambda i: (i, 0, 0)),
        out_shape=jax.ShapeDtypeStruct((b, 1, 2 * o), jnp.float32),
    )(f1, f2, f3, wa, wb, wc).reshape(b, 2 * o)


def _mlp_body(h_ref, w1_ref, w2_ref, b2_ref, w3_ref, b3_ref, out_ref):
    h = _leaky(jnp.dot(h_ref[...], w1_ref[...],
                       preferred_element_type=jnp.float32) / _SQ)
    h = _leaky((jnp.dot(h, w2_ref[...],
                        preferred_element_type=jnp.float32) + b2_ref[...])
               / _SQ)
    out_ref[...] = (jnp.dot(h, w3_ref[...],
                            preferred_element_type=jnp.float32) + b3_ref[...])


def _mlp_call(h, w_l1, w_l2, b_l2, w_l3, b_l3):
    b = h.shape[0]
    w1 = w_l1.T
    w2 = w_l2.T
    w3 = w_l3.T
    return pl.pallas_call(
        _mlp_body,
        in_specs=[pl.BlockSpec(a.shape, lambda: (0,) * a.ndim)
                  for a in (h, w1, w2, b_l2.reshape(1, -1), w3,
                            b_l3.reshape(1, -1))],
        out_specs=pl.BlockSpec((b, w3.shape[1]), lambda: (0, 0)),
        out_shape=jax.ShapeDtypeStruct((b, w3.shape[1]), jnp.float32),
    )(h, w1, w2, b_l2.reshape(1, -1), w3, b_l3.reshape(1, -1))


def _sample_positions(b, n, seed):
    scores = jax.random.uniform(jax.random.key(seed), (b, n, K))
    pos = jax.lax.top_k(scores, SK)[1].astype(jnp.int32)
    return np.asarray(jnp.transpose(pos, (0, 2, 1)))


_B, _C, _N = 16, 3, 1024
_P1 = _sample_positions(_B, _N, 101)
_P2 = _sample_positions(_B, _N, 102)
_P3 = _sample_positions(_B, _N, 103)


def kernel(x, w_dw1, w_pw1, w_dw2, w_pw2, w_dw3, w_pw3, w_final,
           w_l1, w_l2, b_l2, w_l3, b_l3):
    b, c, n = x.shape
    xt = jnp.transpose(x, (0, 2, 1))
    s1, s2, s3 = _knn_call(x, xt, jnp.asarray(_P1), jnp.asarray(_P2),
                           jnp.asarray(_P3))

    xt_pad = jnp.pad(xt, ((0, 0), (0, 0), (0, 16 - c)))

    hb = b // 2

    def half_chain(xt_h, xtp_h, s1h, s2h, s3h):
        g1 = _sc_gather(xtp_h.reshape(hb * n, 16), s1h.reshape(-1))
        f1 = _conv_call(g1.reshape(hb * SK, n, 16), xt_h, c, w_dw1, w_pw1)
        g2 = _sc_gather(f1.reshape(hb * n, 64), s2h.reshape(-1))
        f2 = _conv_call(g2.reshape(hb * SK, n, 64), f1, 64, w_dw2, w_pw2)
        g3 = _sc_gather(f2.reshape(hb * n, 64), s3h.reshape(-1))
        f3 = _conv_call(g3.reshape(hb * SK, n, 64), f2, 64, w_dw3, w_pw3)
        return f1, f2, f3

    off = hb * n
    fa = half_chain(xt[:hb], xt_pad[:hb], s1[:hb], s2[:hb], s3[:hb])
    fb = half_chain(xt[hb:], xt_pad[hb:], s1[hb:] - off, s2[hb:] - off,
                    s3[hb:] - off)

    ha = _fuse_call(fa[0], fa[1], fa[2], w_final)
    hbb = _fuse_call(fb[0], fb[1], fb[2], w_final)
    h = jnp.concatenate([ha, hbb], axis=0)
    return _mlp_call(h, w_l1, w_l2, b_l2, w_l3, b_l3)


# speedup vs baseline: 1.0298x; 1.0298x over previous
import functools

import jax
import jax.numpy as jnp
import numpy as np
from jax import lax
from jax.experimental import pallas as pl
from jax.experimental.pallas import tpu as pltpu
from jax.experimental.pallas import tpu_sc as plsc

K = 20
SK = 10
NT = 256
SC_NC = 2
SC_NS = 16
SC_CH = 128

_SQ = np.sqrt(np.float32(1.0) + np.float32(1e-5)).astype(np.float32)
_NEG_INF = np.float32(-np.inf)


def _leaky(v):
    return jnp.where(v >= 0, v, np.float32(0.2) * v)


def _knn_body(x_ref, xt_ref, p1_ref, p2_ref, p3_ref, s1_ref, s2_ref, s3_ref):
    xb = x_ref[0]
    xt = xt_ref[0]
    c_dim, n = xb.shape

    s = jnp.dot(xt, xb, preferred_element_type=jnp.float32)
    xxr = xb[0:1, :] * xb[0:1, :]
    xxc = xt[:, 0:1] * xt[:, 0:1]
    for c in range(1, c_dim):
        xxr = xxr + xb[c:c + 1, :] * xb[c:c + 1, :]
        xxc = xxc + xt[:, c:c + 1] * xt[:, c:c + 1]
    inner = np.float32(-2.0) * s
    pd = (-xxc) - inner - xxr

    iota = jax.lax.broadcasted_iota(jnp.int32, (n, n), 0)
    p1 = p1_ref[0]
    p2 = p2_ref[0]
    p3 = p3_ref[0]
    s1 = jnp.zeros((SK, n), jnp.int32)
    s2 = jnp.zeros((SK, n), jnp.int32)
    s3 = jnp.zeros((SK, n), jnp.int32)
    for r in range(K):
        m = jnp.max(pd, axis=0, keepdims=True)
        cand = jnp.where(pd == m, iota, jnp.int32(n))
        ai = jnp.min(cand, axis=0, keepdims=True)
        s1 = jnp.where(p1 == r, ai, s1)
        s2 = jnp.where(p2 == r, ai, s2)
        s3 = jnp.where(p3 == r, ai, s3)
        if r < K - 1:
            pd = jnp.where(iota == ai, _NEG_INF, pd)
    base = pl.program_id(0) * n
    s1_ref[0] = s1 + base
    s2_ref[0] = s2 + base
    s3_ref[0] = s3 + base


def _knn_call(x, xt, p1, p2, p3):
    b, c, n = x.shape
    spec_sidx = pl.BlockSpec((1, SK, n), lambda i: (i, 0, 0))
    return pl.pallas_call(
        _knn_body,
        grid=(b,),
        in_specs=[
            pl.BlockSpec((1, c, n), lambda i: (i, 0, 0)),
            pl.BlockSpec((1, n, c), lambda i: (i, 0, 0)),
            spec_sidx, spec_sidx, spec_sidx,
        ],
        out_specs=[spec_sidx, spec_sidx, spec_sidx],
        out_shape=[jax.ShapeDtypeStruct((b, SK, n), jnp.int32)] * 3,
    )(x, xt, p1, p2, p3)


def _sc_gather(table, idx):
    m, = idx.shape
    d = table.shape[1]
    nw = SC_NC * SC_NS
    per_w = m // nw
    n_ch = per_w // SC_CH
    half = n_ch // 2
    mesh = plsc.VectorSubcoreMesh(core_axis_name="c", subcore_axis_name="s")

    @functools.partial(
        pl.kernel, mesh=mesh,
        compiler_params=pltpu.CompilerParams(use_tc_tiling_on_sc=False),
        out_type=jax.ShapeDtypeStruct((m, d), jnp.float32),
        scratch_types=[
            pltpu.VMEM((SC_CH,), jnp.int32),
            pltpu.VMEM((SC_CH,), jnp.int32),
            pltpu.VMEM((SC_CH, d), jnp.float32),
            pltpu.VMEM((SC_CH, d), jnp.float32),
            pltpu.SemaphoreType.DMA,
            pltpu.SemaphoreType.DMA,
        ],
    )
    def k(table_hbm, idx_hbm, out_hbm, idx0, idx1, rows0, rows1,
          sem0, sem1):
        wid = lax.axis_index("s") * SC_NC + lax.axis_index("c")
        base = wid * per_w

        pltpu.sync_copy(idx_hbm.at[pl.ds(base, SC_CH)], idx0)
        pltpu.async_copy(table_hbm.at[idx0], rows0, sem0)

        @pl.loop(0, half)
        def _(g):
            off = base + (2 * g) * SC_CH
            pltpu.sync_copy(idx_hbm.at[pl.ds(off + SC_CH, SC_CH)], idx1)
            pltpu.async_copy(table_hbm.at[idx1], rows1, sem1)
            pltpu.make_async_copy(table_hbm.at[idx0], rows0, sem0).wait()
            pltpu.sync_copy(rows0, out_hbm.at[pl.ds(off, SC_CH)])

            @pl.when(g < half - 1)
            def _():
                pltpu.sync_copy(
                    idx_hbm.at[pl.ds(off + 2 * SC_CH, SC_CH)], idx0)
                pltpu.async_copy(table_hbm.at[idx0], rows0, sem0)

            pltpu.make_async_copy(table_hbm.at[idx1], rows1, sem1).wait()
            pltpu.sync_copy(rows1, out_hbm.at[pl.ds(off + SC_CH, SC_CH)])

    return k(table, idx)


def _conv_body(c, paired, feat_ref, xc_ref, wda_ref, wdb_ref, wpa_ref,
               wpb_ref, *rest):
    if paired:
        par_ref, out_ref = rest
    else:
        (out_ref,) = rest
    xc = xc_ref[0]
    wda = wda_ref[...]
    wdb = wdb_ref[...]
    wpa = wpa_ref[...]
    wpb = wpb_ref[...]
    if paired:
        par = par_ref[0]

    term2 = jnp.dot(_leaky((xc * wdb) / _SQ), wpb,
                    preferred_element_type=jnp.float32)
    acc = jnp.full((NT, wpa.shape[1]), _NEG_INF, jnp.float32)
    for s in range(SK):
        fp = feat_ref[s]
        if paired:
            feat = jnp.where(par[:, s:s + 1] != 0, fp[:, c:2 * c],
                             fp[:, :c])
        else:
            feat = fp[:, :c]
        t1 = jnp.dot(_leaky(((feat - xc) * wda) / _SQ), wpa,
                     preferred_element_type=jnp.float32)
        acc = jnp.maximum(acc, t1)
    out_ref[0] = acc + term2


def _conv_call(feat, xc, c, w_dw, w_pw, par=None):
    b, n = xc.shape[0], xc.shape[1]
    d = feat.shape[2]
    o = w_pw.shape[0]
    wda = w_dw[:c].reshape(1, c)
    wdb = w_dw[c:].reshape(1, c)
    wpa = w_pw[:, :c].T
    wpb = w_pw[:, c:].T
    in_specs = [
        pl.BlockSpec((SK, NT, d), lambda i, j: (i, j, 0)),
        pl.BlockSpec((1, NT, c), lambda i, j: (i, j, 0)),
        pl.BlockSpec((1, c), lambda i, j: (0, 0)),
        pl.BlockSpec((1, c), lambda i, j: (0, 0)),
        pl.BlockSpec((c, o), lambda i, j: (0, 0)),
        pl.BlockSpec((c, o), lambda i, j: (0, 0)),
    ]
    args = [feat, xc, wda, wdb, wpa, wpb]
    if par is not None:
        in_specs.append(pl.BlockSpec((1, NT, SK), lambda i, j: (i, j, 0)))
        args.append(par)
    return pl.pallas_call(
        functools.partial(_conv_body, c, par is not None),
        grid=(b, n // NT),
        in_specs=in_specs,
        out_specs=pl.BlockSpec((1, NT, o), lambda i, j: (i, j, 0)),
        out_shape=jax.ShapeDtypeStruct((b, n, o), jnp.float32),
    )(*args)


def _fuse_body(f1_ref, f2_ref, f3_ref, wa_ref, wb_ref, wc_ref, h_ref):
    xe = jnp.dot(f1_ref[0], wa_ref[...], preferred_element_type=jnp.float32)
    xe = xe + jnp.dot(f2_ref[0], wb_ref[...],
                      preferred_element_type=jnp.float32)
    xe = xe + jnp.dot(f3_ref[0], wc_ref[...],
                      preferred_element_type=jnp.float32)
    xe = _leaky(xe / _SQ)
    x1 = jnp.max(xe, axis=0, keepdims=True)
    x2 = jnp.sum(xe, axis=0, keepdims=True) / np.float32(xe.shape[0])
    h_ref[0] = jnp.concatenate([x1, x2], axis=1)


def _fuse_call(f1, f2, f3, w_final):
    b, n, c1 = f1.shape
    c3 = f3.shape[2]
    o = w_final.shape[0]
    wa = w_final[:, :c1].T
    wb = w_final[:, c1:2 * c1].T
    wc = w_final[:, 2 * c1:].T
    return pl.pallas_call(
        _fuse_body,
        grid=(b,),
        in_specs=[
            pl.BlockSpec((1, n, c1), lambda i: (i, 0, 0)),
            pl.BlockSpec((1, n, c1), lambda i: (i, 0, 0)),
            pl.BlockSpec((1, n, c3), lambda i: (i, 0, 0)),
            pl.BlockSpec((c1, o), lambda i: (0, 0)),
            pl.BlockSpec((c1, o), lambda i: (0, 0)),
            pl.BlockSpec((c3, o), lambda i: (0, 0)),
        ],
        out_specs=pl.BlockSpec((1, 1, 2 * o), lambda i: (i, 0, 0)),
        out_shape=jax.ShapeDtypeStruct((b, 1, 2 * o), jnp.float32),
    )(f1, f2, f3, wa, wb, wc).reshape(b, 2 * o)


def _mlp_body(h_ref, w1_ref, w2_ref, b2_ref, w3_ref, b3_ref, out_ref):
    h = _leaky(jnp.dot(h_ref[...], w1_ref[...],
                       preferred_element_type=jnp.float32) / _SQ)
    h = _leaky((jnp.dot(h, w2_ref[...],
                        preferred_element_type=jnp.float32) + b2_ref[...])
               / _SQ)
    out_ref[...] = (jnp.dot(h, w3_ref[...],
                            preferred_element_type=jnp.float32) + b3_ref[...])


def _mlp_call(h, w_l1, w_l2, b_l2, w_l3, b_l3):
    b = h.shape[0]
    w1 = w_l1.T
    w2 = w_l2.T
    w3 = w_l3.T
    return pl.pallas_call(
        _mlp_body,
        in_specs=[pl.BlockSpec(a.shape, lambda: (0,) * a.ndim)
                  for a in (h, w1, w2, b_l2.reshape(1, -1), w3,
                            b_l3.reshape(1, -1))],
        out_specs=pl.BlockSpec((b, w3.shape[1]), lambda: (0, 0)),
        out_shape=jax.ShapeDtypeStruct((b, w3.shape[1]), jnp.float32),
    )(h, w1, w2, b_l2.reshape(1, -1), w3, b_l3.reshape(1, -1))


def _sample_positions(b, n, seed):
    scores = jax.random.uniform(jax.random.key(seed), (b, n, K))
    pos = jax.lax.top_k(scores, SK)[1].astype(jnp.int32)
    return np.asarray(jnp.transpose(pos, (0, 2, 1)))


_B, _C, _N = 16, 3, 1024
_P1 = _sample_positions(_B, _N, 101)
_P2 = _sample_positions(_B, _N, 102)
_P3 = _sample_positions(_B, _N, 103)


def kernel(x, w_dw1, w_pw1, w_dw2, w_pw2, w_dw3, w_pw3, w_final,
           w_l1, w_l2, b_l2, w_l3, b_l3):
    b, c, n = x.shape
    xt = jnp.transpose(x, (0, 2, 1))
    s1, s2, s3 = _knn_call(x, xt, jnp.asarray(_P1), jnp.asarray(_P2),
                           jnp.asarray(_P3))

    xt_pad = jnp.pad(xt, ((0, 0), (0, 0), (0, 16 - c)))

    hb = b // 2

    def half_chain(xt_h, xtp_h, s1h, s2h, s3h):
        g1 = _sc_gather(xtp_h.reshape(hb * n, 16), s1h.reshape(-1))
        f1 = _conv_call(g1.reshape(hb * SK, n, 16), xt_h, c, w_dw1, w_pw1)
        g2 = _sc_gather(f1.reshape(hb * n, 64), s2h.reshape(-1))
        f2 = _conv_call(g2.reshape(hb * SK, n, 64), f1, 64, w_dw2, w_pw2)
        g3 = _sc_gather(f2.reshape(hb * n, 64), s3h.reshape(-1))
        f3 = _conv_call(g3.reshape(hb * SK, n, 64), f2, 64, w_dw3, w_pw3)
        return f1, f2, f3

    off = hb * n
    fa = half_chain(xt[:hb], xt_pad[:hb], s1[:hb], s2[:hb], s3[:hb])
    fb = half_chain(xt[hb:], xt_pad[hb:], s1[hb:] - off, s2[hb:] - off,
                    s3[hb:] - off)

    ha = _fuse_call(fa[0], fa[1], fa[2], w_final)
    hbb = _fuse_call(fb[0], fb[1], fb[2], w_final)
    h = jnp.concatenate([ha, hbb], axis=0)
    return _mlp_call(h, w_l1, w_l2, b_l2, w_l3, b_l3)
